# Initial kernel scaffold; baseline (speedup 1.0000x reference)
import jax
import jax.numpy as jnp
from jax.experimental import pallas as pl


def kernel(tokens, nodes, token_table, node_table):
    raise NotImplementedError("write your pallas kernel here")


# speedup vs baseline: 3.4144x; 3.4144x over previous
import functools

import jax
import jax.numpy as jnp
from jax import lax
from jax.experimental import pallas as pl
from jax.experimental.pallas import tpu as pltpu
from jax.experimental.pallas import tpu_sc as plsc

N_NODES = 50000
SUBTOK = 20
EMB = 128

NC = 2
NS = 16
NW = NC * NS

PER_W = 1568
N_PAD = NW * PER_W
BBLK = 32
NBLK = PER_W // BBLK
ROWS_BLK = BBLK * SUBTOK
KCHUNK = ROWS_BLK // 128

_mesh = plsc.VectorSubcoreMesh(core_axis_name="c", subcore_axis_name="s")


@functools.partial(
    pl.kernel,
    out_type=jax.ShapeDtypeStruct((N_PAD, EMB), jnp.float32),
    mesh=_mesh,
    scratch_types=[
        pltpu.VMEM((ROWS_BLK,), jnp.int32),
        pltpu.VMEM((BBLK,), jnp.int32),
        pltpu.VMEM((ROWS_BLK, EMB), jnp.float32),
        pltpu.VMEM((BBLK, EMB), jnp.float32),
        pltpu.SemaphoreType.DMA,
        pltpu.SemaphoreType.DMA,
    ],
)
def _node_embedding_sc(tokens_hbm, nodes_hbm, token_table, node_table,
                       out_hbm, tok_idx_v, node_idx_v, rows_v, acc_v,
                       sem_rows, sem_acc):
    wid = lax.axis_index("s") * NC + lax.axis_index("c")

    def block_body(blk, _):
        base = wid * PER_W + blk * BBLK
        pltpu.sync_copy(tokens_hbm.at[pl.ds(base * SUBTOK, ROWS_BLK)],
                        tok_idx_v)
        pltpu.sync_copy(nodes_hbm.at[pl.ds(base, BBLK)], node_idx_v)
        node_cp = pltpu.async_copy(node_table.at[node_idx_v], acc_v, sem_acc)
        cps = []
        for k in range(KCHUNK):
            cps.append(pltpu.async_copy(
                token_table.at[tok_idx_v.at[pl.ds(k * 128, 128)]],
                rows_v.at[pl.ds(k * 128, 128)], sem_rows))
        node_cp.wait()
        for cp in cps:
            cp.wait()

        def node_body(i, _):
            for c in range(EMB // 16):
                s = pl.ds(c * 16, 16)
                v = acc_v[i, s]
                for j in range(SUBTOK):
                    v = v + rows_v[i * SUBTOK + j, s]
                acc_v[i, s] = v
            return 0

        lax.fori_loop(0, BBLK, node_body, 0)
        pltpu.sync_copy(acc_v, out_hbm.at[pl.ds(base, BBLK)])
        return 0

    lax.fori_loop(0, NBLK, block_body, 0)


def kernel(tokens, nodes, token_table, node_table):
    tokens = tokens.astype(jnp.int32)
    nodes = nodes.astype(jnp.int32)
    tokens_p = jnp.zeros((N_PAD, SUBTOK), jnp.int32).at[:N_NODES].set(tokens)
    nodes_p = jnp.zeros((N_PAD,), jnp.int32).at[:N_NODES].set(nodes)
    tokens_flat = tokens_p.reshape(N_PAD * SUBTOK)
    out = _node_embedding_sc(tokens_flat, nodes_p, token_table, node_table)
    return out[:N_NODES]


# speedup vs baseline: 5.1898x; 1.5200x over previous
import functools

import jax
import jax.numpy as jnp
from jax import lax
from jax.experimental import pallas as pl
from jax.experimental.pallas import tpu as pltpu
from jax.experimental.pallas import tpu_sc as plsc

N_NODES = 50000
SUBTOK = 20
EMB = 128

NC = 2
NS = 16
NW = NC * NS

PER_W = 1568
N_PAD = NW * PER_W
BBLK = 32
NBLK = PER_W // BBLK
ROWS_BLK = BBLK * SUBTOK

_mesh = plsc.VectorSubcoreMesh(core_axis_name="c", subcore_axis_name="s")


@functools.partial(
    pl.kernel,
    out_type=jax.ShapeDtypeStruct((N_PAD, EMB), jnp.float32),
    mesh=_mesh,
    scratch_types=[
        pltpu.VMEM((ROWS_BLK,), jnp.int32),
        pltpu.VMEM((BBLK,), jnp.int32),
        pltpu.VMEM((BBLK, EMB), jnp.float32),
        pltpu.SemaphoreType.DMA,
        pltpu.SemaphoreType.DMA,
    ],
)
def _node_embedding_sc(tokens_hbm, nodes_hbm, token_table, node_table,
                       out_hbm, tok_idx_v, node_idx_v, acc_v,
                       sem_rows, sem_acc):
    wid = lax.axis_index("s") * NC + lax.axis_index("c")

    def block_body(blk, _):
        base = wid * PER_W + blk * BBLK
        pltpu.sync_copy(tokens_hbm.at[pl.ds(base * SUBTOK, ROWS_BLK)],
                        tok_idx_v)
        pltpu.sync_copy(nodes_hbm.at[pl.ds(base, BBLK)], node_idx_v)
        pltpu.async_copy(node_table.at[node_idx_v], acc_v, sem_acc).wait()
        cps = []
        for j in range(SUBTOK):
            cps.append(pltpu.async_copy(
                token_table.at[tok_idx_v.at[pl.ds(j * BBLK, BBLK)]],
                acc_v, sem_rows, add=True))
        for cp in cps:
            cp.wait()
        pltpu.sync_copy(acc_v, out_hbm.at[pl.ds(base, BBLK)])
        return 0

    lax.fori_loop(0, NBLK, block_body, 0)


def kernel(tokens, nodes, token_table, node_table):
    tokens = tokens.astype(jnp.int32)
    nodes = nodes.astype(jnp.int32)
    tokens_p = jnp.zeros((N_PAD, SUBTOK), jnp.int32).at[:N_NODES].set(tokens)
    nodes_p = jnp.zeros((N_PAD,), jnp.int32).at[:N_NODES].set(nodes)
    tokens_flat = (tokens_p.reshape(N_PAD // BBLK, BBLK, SUBTOK)
                   .transpose(0, 2, 1)
                   .reshape(N_PAD * SUBTOK))
    out = _node_embedding_sc(tokens_flat, nodes_p, token_table, node_table)
    return out[:N_NODES]


# speedup vs baseline: 5.9071x; 1.1382x over previous
import functools

import jax
import jax.numpy as jnp
from jax import lax
from jax.experimental import pallas as pl
from jax.experimental.pallas import tpu as pltpu
from jax.experimental.pallas import tpu_sc as plsc

N_NODES = 50000
SUBTOK = 20
EMB = 128

NC = 2
NS = 16
NW = NC * NS

PER_W = 1568
N_PAD = NW * PER_W
HALF = PER_W // 2
CH = 112
NCH = HALF // CH
IDX_HALF = HALF * SUBTOK

_mesh = plsc.VectorSubcoreMesh(core_axis_name="c", subcore_axis_name="s")


@functools.partial(
    pl.kernel,
    out_type=jax.ShapeDtypeStruct((N_PAD, EMB), jnp.float32),
    mesh=_mesh,
    scratch_types=[
        pltpu.VMEM((IDX_HALF,), jnp.int32),
        pltpu.VMEM((HALF,), jnp.int32),
        pltpu.VMEM((HALF, EMB), jnp.float32),
        pltpu.SemaphoreType.DMA,
        pltpu.SemaphoreType.DMA,
    ],
)
def _node_embedding_sc(tokens_hbm, nodes_hbm, token_table, node_table,
                       out_hbm, tok_idx_v, node_idx_v, acc_v,
                       sem_add, sem_init):
    wid = lax.axis_index("s") * NC + lax.axis_index("c")

    def half_body(h, _):
        base = wid * PER_W + h * HALF
        pltpu.sync_copy(tokens_hbm.at[pl.ds(base * SUBTOK, IDX_HALF)],
                        tok_idx_v)
        pltpu.sync_copy(nodes_hbm.at[pl.ds(base, HALF)], node_idx_v)
        init_cps = []
        for c in range(NCH):
            s = pl.ds(c * CH, CH)
            init_cps.append(pltpu.async_copy(
                node_table.at[node_idx_v.at[s]], acc_v.at[s], sem_init))
        for cp in init_cps:
            cp.wait()

        def sub_body(j, _):
            cps = []
            for c in range(NCH):
                cps.append(pltpu.async_copy(
                    token_table.at[
                        tok_idx_v.at[pl.ds(c * (CH * SUBTOK) + j * CH, CH)]],
                    acc_v.at[pl.ds(c * CH, CH)], sem_add, add=True))
            for cp in cps:
                cp.wait()
            return 0

        lax.fori_loop(0, SUBTOK, sub_body, 0)
        pltpu.sync_copy(acc_v, out_hbm.at[pl.ds(base, HALF)])
        return 0

    lax.fori_loop(0, 2, half_body, 0)


def kernel(tokens, nodes, token_table, node_table):
    tokens = tokens.astype(jnp.int32)
    nodes = nodes.astype(jnp.int32)
    tokens_p = jnp.zeros((N_PAD, SUBTOK), jnp.int32).at[:N_NODES].set(tokens)
    nodes_p = jnp.zeros((N_PAD,), jnp.int32).at[:N_NODES].set(nodes)
    tokens_flat = (tokens_p.reshape(N_PAD // CH, CH, SUBTOK)
                   .transpose(0, 2, 1)
                   .reshape(N_PAD * SUBTOK))
    out = _node_embedding_sc(tokens_flat, nodes_p, token_table, node_table)
    return out[:N_NODES]


# speedup vs baseline: 6.2777x; 1.0627x over previous
import functools

import jax
import jax.numpy as jnp
from jax import lax
from jax.experimental import pallas as pl
from jax.experimental.pallas import tpu as pltpu
from jax.experimental.pallas import tpu_sc as plsc

N_NODES = 50000
SUBTOK = 20
EMB = 128

NC = 2
NS = 16
NW = NC * NS

PER_W = 1568
N_PAD = NW * PER_W
HALF = PER_W // 2
CH = 112
NCH = HALF // CH
IDX_HALF = HALF * SUBTOK

_mesh = plsc.VectorSubcoreMesh(core_axis_name="c", subcore_axis_name="s")


@functools.partial(
    pl.kernel,
    out_type=jax.ShapeDtypeStruct((N_PAD, EMB), jnp.float32),
    mesh=_mesh,
    scratch_types=[
        pltpu.VMEM((IDX_HALF,), jnp.int32),
        pltpu.VMEM((HALF,), jnp.int32),
        pltpu.VMEM((HALF, EMB), jnp.float32),
        pltpu.SemaphoreType.DMA,
        pltpu.SemaphoreType.DMA,
    ],
)
def _node_embedding_sc(tokens_hbm, nodes_hbm, token_table, node_table,
                       out_hbm, tok_idx_v, node_idx_v, acc_v,
                       sem_add, sem_init):
    wid = lax.axis_index("s") * NC + lax.axis_index("c")

    def half_body(h, _):
        base = wid * PER_W + h * HALF
        pltpu.sync_copy(tokens_hbm.at[pl.ds(base * SUBTOK, IDX_HALF)],
                        tok_idx_v)
        pltpu.sync_copy(nodes_hbm.at[pl.ds(base, HALF)], node_idx_v)
        init_cps = []
        for c in range(NCH):
            s = pl.ds(c * CH, CH)
            init_cps.append(pltpu.async_copy(
                node_table.at[node_idx_v.at[s]], acc_v.at[s], sem_init))
        for cp in init_cps:
            cp.wait()

        def sub_body(j, _):
            for c in range(NCH):
                pltpu.async_copy(
                    token_table.at[
                        tok_idx_v.at[pl.ds(c * (CH * SUBTOK) + j * CH, CH)]],
                    acc_v.at[pl.ds(c * CH, CH)], sem_add, add=True)
            return 0

        lax.fori_loop(0, SUBTOK, sub_body, 0)

        def drain_body(j, _):
            pltpu.make_async_copy(
                token_table.at[pl.ds(0, HALF)], acc_v, sem_add).wait()
            return 0

        lax.fori_loop(0, SUBTOK, drain_body, 0)
        pltpu.sync_copy(acc_v, out_hbm.at[pl.ds(base, HALF)])
        return 0

    lax.fori_loop(0, 2, half_body, 0)


def kernel(tokens, nodes, token_table, node_table):
    tokens = tokens.astype(jnp.int32)
    nodes = nodes.astype(jnp.int32)
    tokens_p = jnp.zeros((N_PAD, SUBTOK), jnp.int32).at[:N_NODES].set(tokens)
    nodes_p = jnp.zeros((N_PAD,), jnp.int32).at[:N_NODES].set(nodes)
    tokens_flat = (tokens_p.reshape(N_PAD // CH, CH, SUBTOK)
                   .transpose(0, 2, 1)
                   .reshape(N_PAD * SUBTOK))
    out = _node_embedding_sc(tokens_flat, nodes_p, token_table, node_table)
    return out[:N_NODES]
